# baseline (speedup 1.0000x reference)
import functools

import jax
import jax.numpy as jnp
from jax import lax
from jax.experimental import pallas as pl
from jax.experimental.pallas import tpu as pltpu
from jax.experimental.pallas import tpu_sc as plsc

HID = 1024
INT = 512
NE = 8
T = 2048
TOP_SCALE = 2.5
TILE = 256
S_PAD = (NE - 1) * TILE + 2 * T
NT_R = S_PAD // TILE
NT = NT_R
NSUB = 32
CHUNK_B = T // NSUB
CHUNK_D = 32


def _sigmoid(x):
    return 1.0 / (1.0 + jnp.exp(-x))


def _silu(x):
    return x * _sigmoid(x)


def _dotT(a, b):
    return jax.lax.dot_general(a, b, (((1,), (1,)), ((), ())),
                               preferred_element_type=jnp.float32)


def _router_body(x_ref, gw_ref, w0_ref, w1_ref, p0_ref, p1_ref, te_ref):
    x = x_ref[...]
    logits = jax.lax.dot_general(gw_ref[...], x, (((1,), (1,)), ((), ())),
                                 preferred_element_type=jnp.float32)
    s = _sigmoid(logits)
    ei = jax.lax.broadcasted_iota(jnp.int32, s.shape, 0)
    m1 = jnp.max(s, axis=0, keepdims=True)
    f1 = jnp.min(jnp.where(s == m1, ei, NE), axis=0, keepdims=True)
    mask1 = ei == f1
    sx = jnp.where(mask1, -1.0, s)
    m2 = jnp.max(sx, axis=0, keepdims=True)
    f2 = jnp.min(jnp.where(sx == m2, ei, NE), axis=0, keepdims=True)
    mask2 = ei == f2
    scale = TOP_SCALE / (m1 + m2 + 1e-20)
    w0_ref[...] = jnp.broadcast_to(jnp.reshape(m1 * scale, (T, 1)), (T, 128))
    w1_ref[...] = jnp.broadcast_to(jnp.reshape(m2 * scale, (T, 1)), (T, 128))

    cnt = (mask1 | mask2).astype(jnp.float32)
    t_i = jax.lax.broadcasted_iota(jnp.int32, (T, T), 0)
    t_j = jax.lax.broadcasted_iota(jnp.int32, (T, T), 1)
    ltri = (t_i <= t_j).astype(jnp.float32)
    csum = jax.lax.dot_general(cnt, ltri, (((1,), (0,)), ((), ())),
                               preferred_element_type=jnp.float32)
    cex = csum - cnt
    counts = csum[:, T - 1:T]
    padded = jnp.floor((counts + (TILE - 1)) * (1.0 / TILE)) * TILE
    e_i = jax.lax.broadcasted_iota(jnp.int32, (NE, NE), 0)
    e_j = jax.lax.broadcasted_iota(jnp.int32, (NE, NE), 1)
    lex = (e_j < e_i).astype(jnp.float32)
    offs = jax.lax.dot_general(lex, padded, (((1,), (0,)), ((), ())),
                               preferred_element_type=jnp.float32)
    pos_base = offs + cex
    p0 = jnp.sum(jnp.where(mask1, pos_base, 0.0), axis=0, keepdims=True)
    p1 = jnp.sum(jnp.where(mask2, pos_base, 0.0), axis=0, keepdims=True)
    p0_ref[...] = p0.astype(jnp.int32)
    p1_ref[...] = p1.astype(jnp.int32)

    jt = jax.lax.broadcasted_iota(jnp.int32, (NE, NT), 1).astype(jnp.float32)
    e_col = jax.lax.broadcasted_iota(jnp.int32, (NE, NT), 0)
    lo = offs * (1.0 / TILE)
    hi = (offs + padded) * (1.0 / TILE)
    sel = (jt >= lo) & (jt < hi)
    te = jnp.sum(jnp.where(sel, e_col, 0), axis=0, keepdims=True)
    jt_i = jax.lax.broadcasted_iota(jnp.int32, (1, NT), 1)
    te_ref[...] = jnp.where(jt_i >= NT_R, NE, te)


@jax.jit
def _stage_a(x2d, gate_weight):
    outs = pl.pallas_call(
        _router_body,
        grid=(1,),
        in_specs=[
            pl.BlockSpec((T, HID), lambda i: (0, 0)),
            pl.BlockSpec((NE, HID), lambda i: (0, 0)),
        ],
        out_specs=[
            pl.BlockSpec((T, 128), lambda i: (0, 0)),
            pl.BlockSpec((T, 128), lambda i: (0, 0)),
            pl.BlockSpec((1, T), lambda i: (0, 0)),
            pl.BlockSpec((1, T), lambda i: (0, 0)),
            pl.BlockSpec((1, NT), lambda i: (0, 0)),
        ],
        out_shape=[
            jax.ShapeDtypeStruct((T, 128), jnp.float32),
            jax.ShapeDtypeStruct((T, 128), jnp.float32),
            jax.ShapeDtypeStruct((1, T), jnp.int32),
            jax.ShapeDtypeStruct((1, T), jnp.int32),
            jax.ShapeDtypeStruct((1, NT), jnp.int32),
        ],
    )(x2d, gate_weight)
    w0, w1, p0, p1, te = outs
    return (w0, w1, p0.reshape(T), p1.reshape(T), te.reshape(NT))


def _scatter_body(x_hbm, p0_hbm, p1_hbm, w0_hbm, w1_hbm, xs_hbm, ws_hbm,
                  xrows, p0v, p1v, w0r, w1r, sem):
    wid = lax.axis_index("s") * 2 + lax.axis_index("c")
    base = wid * CHUNK_B
    pltpu.sync_copy(x_hbm.at[pl.ds(base, CHUNK_B)], xrows)
    pltpu.sync_copy(p0_hbm.at[pl.ds(base, CHUNK_B)], p0v)
    pltpu.sync_copy(p1_hbm.at[pl.ds(base, CHUNK_B)], p1v)
    pltpu.sync_copy(w0_hbm.at[pl.ds(base, CHUNK_B)], w0r)
    pltpu.sync_copy(w1_hbm.at[pl.ds(base, CHUNK_B)], w1r)
    a = pltpu.async_copy(xrows, xs_hbm.at[p0v], sem)
    b = pltpu.async_copy(xrows, xs_hbm.at[p1v], sem)
    c = pltpu.async_copy(w0r, ws_hbm.at[p0v], sem)
    d = pltpu.async_copy(w1r, ws_hbm.at[p1v], sem)
    a.wait()
    b.wait()
    c.wait()
    d.wait()


@jax.jit
def _stage_b(x2d, p0, p1, w0, w1):
    mesh = plsc.VectorSubcoreMesh(core_axis_name="c", subcore_axis_name="s")
    return pl.kernel(
        _scatter_body,
        mesh=mesh,
        out_type=[
            jax.ShapeDtypeStruct((S_PAD, HID), jnp.float32),
            jax.ShapeDtypeStruct((S_PAD, 128), jnp.float32),
        ],
        scratch_types=[
            pltpu.VMEM((CHUNK_B, HID), jnp.float32),
            pltpu.VMEM((CHUNK_B,), jnp.int32),
            pltpu.VMEM((CHUNK_B,), jnp.int32),
            pltpu.VMEM((CHUNK_B, 128), jnp.float32),
            pltpu.VMEM((CHUNK_B, 128), jnp.float32),
            pltpu.SemaphoreType.DMA,
        ],
    )(x2d, p0, p1, w0, w1)


def _gmm_body(te_ref, xs_ref, ws_ref, egw_ref, euw_ref, edw_ref, os_ref):
    xb = xs_ref[...]
    g = _dotT(xb, egw_ref[0])
    u = _dotT(xb, euw_ref[0])
    h = _silu(g) * u
    os_ref[...] = _dotT(h, edw_ref[0]) * ws_ref[:, 0:1]


@jax.jit
def _stage_c1(te, xs, ws, egw, euw, edw):
    grid_spec = pltpu.PrefetchScalarGridSpec(
        num_scalar_prefetch=1,
        grid=(NT_R,),
        in_specs=[
            pl.BlockSpec((TILE, HID), lambda j, s: (j, 0)),
            pl.BlockSpec((TILE, 128), lambda j, s: (j, 0)),
            pl.BlockSpec((1, INT, HID), lambda j, s: (s[j], 0, 0)),
            pl.BlockSpec((1, INT, HID), lambda j, s: (s[j], 0, 0)),
            pl.BlockSpec((1, HID, INT), lambda j, s: (s[j], 0, 0)),
        ],
        out_specs=pl.BlockSpec((TILE, HID), lambda j, s: (j, 0)),
    )
    return pl.pallas_call(
        _gmm_body,
        grid_spec=grid_spec,
        out_shape=jax.ShapeDtypeStruct((S_PAD, HID), jnp.float32),
    )(te, xs, ws, egw, euw, edw)


STILE = 256


def _shared_body(x_ref, sgw_ref, suw_ref, sdw_ref, os_ref):
    xb = x_ref[...]
    g = _dotT(xb, sgw_ref[...])
    u = _dotT(xb, suw_ref[...])
    h = _silu(g) * u
    os_ref[...] = _dotT(h, sdw_ref[...])


@jax.jit
def _stage_c2(x2d, sgw, suw, sdw):
    return pl.pallas_call(
        _shared_body,
        grid=(T // STILE,),
        in_specs=[
            pl.BlockSpec((STILE, HID), lambda t: (t, 0)),
            pl.BlockSpec((INT, HID), lambda t: (0, 0)),
            pl.BlockSpec((INT, HID), lambda t: (0, 0)),
            pl.BlockSpec((HID, INT), lambda t: (0, 0)),
        ],
        out_specs=pl.BlockSpec((STILE, HID), lambda t: (t, 0)),
        out_shape=jax.ShapeDtypeStruct((T, HID), jnp.float32),
    )(x2d, sgw, suw, sdw)


def _combine_body(os_hbm, sh_hbm, p0_hbm, p1_hbm, out_hbm,
                  p0v, p1v, r0, r1, rs, sem):
    wid = lax.axis_index("s") * 2 + lax.axis_index("c")

    for c in range(CHUNK_B // CHUNK_D):
        base = wid * CHUNK_B + c * CHUNK_D
        pltpu.sync_copy(p0_hbm.at[pl.ds(base, CHUNK_D)], p0v)
        pltpu.sync_copy(p1_hbm.at[pl.ds(base, CHUNK_D)], p1v)
        a = pltpu.async_copy(os_hbm.at[p0v], r0, sem)
        b = pltpu.async_copy(os_hbm.at[p1v], r1, sem)
        pltpu.sync_copy(sh_hbm.at[pl.ds(base, CHUNK_D)], rs)
        a.wait()
        b.wait()

        def tok_body(i, carry):
            def col_body(k, carry2):
                for u in range(4):
                    sl = pl.ds((k * 4 + u) * 16, 16)
                    r0[i, sl] = r0[i, sl] + r1[i, sl] + rs[i, sl]
                return carry2

            return lax.fori_loop(0, HID // 64, col_body, carry)

        lax.fori_loop(0, CHUNK_D, tok_body, 0)
        pltpu.sync_copy(r0, out_hbm.at[pl.ds(base, CHUNK_D)])


@jax.jit
def _stage_d(osorted, shared_out, p0, p1):
    mesh = plsc.VectorSubcoreMesh(core_axis_name="c", subcore_axis_name="s")
    return pl.kernel(
        _combine_body,
        mesh=mesh,
        out_type=jax.ShapeDtypeStruct((T, HID), jnp.float32),
        scratch_types=[
            pltpu.VMEM((CHUNK_D,), jnp.int32),
            pltpu.VMEM((CHUNK_D,), jnp.int32),
            pltpu.VMEM((CHUNK_D, HID), jnp.float32),
            pltpu.VMEM((CHUNK_D, HID), jnp.float32),
            pltpu.VMEM((CHUNK_D, HID), jnp.float32),
            pltpu.SemaphoreType.DMA,
        ],
    )(osorted, shared_out, p0, p1)


def kernel(hidden_states, gate_weight, e_score_correction_bias,
           expert_gate_w, expert_up_w, expert_down_w,
           shared_gate_w, shared_up_w, shared_down_w):
    orig_shape = hidden_states.shape
    x2d = hidden_states.reshape(-1, orig_shape[-1])
    w0, w1, p0, p1, te = _stage_a(x2d, gate_weight)
    xs, ws = _stage_b(x2d, p0, p1, w0, w1)
    osorted = _stage_c1(te, xs, ws, expert_gate_w, expert_up_w,
                        expert_down_w)
    shared_out = _stage_c2(x2d, shared_gate_w, shared_up_w, shared_down_w)
    out = _stage_d(osorted, shared_out, p0, p1)
    return out.reshape(orig_shape)


# speedup vs baseline: 1.0293x; 1.0293x over previous
import functools

import jax
import jax.numpy as jnp
from jax import lax
from jax.experimental import pallas as pl
from jax.experimental.pallas import tpu as pltpu
from jax.experimental.pallas import tpu_sc as plsc

HID = 1024
INT = 512
NE = 8
T = 2048
TOP_SCALE = 2.5
TILE = 256
S_PAD = (NE - 1) * TILE + 2 * T
NT_R = S_PAD // TILE
NT = NT_R
NSUB = 32
CHUNK_B = T // NSUB
CHUNK_D = 32


def _sigmoid(x):
    return 1.0 / (1.0 + jnp.exp(-x))


def _silu(x):
    return x * _sigmoid(x)


def _dotT(a, b):
    return jax.lax.dot_general(a, b, (((1,), (1,)), ((), ())),
                               preferred_element_type=jnp.float32)


def _router_body(x_ref, gw_ref, w0_ref, w1_ref, p0_ref, p1_ref, te_ref):
    x = x_ref[...]
    logits = jax.lax.dot_general(gw_ref[...], x, (((1,), (1,)), ((), ())),
                                 preferred_element_type=jnp.float32)
    s = _sigmoid(logits)
    ei = jax.lax.broadcasted_iota(jnp.int32, s.shape, 0)
    m1 = jnp.max(s, axis=0, keepdims=True)
    f1 = jnp.min(jnp.where(s == m1, ei, NE), axis=0, keepdims=True)
    mask1 = ei == f1
    sx = jnp.where(mask1, -1.0, s)
    m2 = jnp.max(sx, axis=0, keepdims=True)
    f2 = jnp.min(jnp.where(sx == m2, ei, NE), axis=0, keepdims=True)
    mask2 = ei == f2
    scale = TOP_SCALE / (m1 + m2 + 1e-20)
    w0_ref[...] = jnp.broadcast_to(jnp.reshape(m1 * scale, (T, 1)), (T, 128))
    w1_ref[...] = jnp.broadcast_to(jnp.reshape(m2 * scale, (T, 1)), (T, 128))

    cnt = (mask1 | mask2).astype(jnp.float32)
    t_i = jax.lax.broadcasted_iota(jnp.int32, (T, T), 0)
    t_j = jax.lax.broadcasted_iota(jnp.int32, (T, T), 1)
    ltri = (t_i <= t_j).astype(jnp.float32)
    csum = jax.lax.dot_general(cnt, ltri, (((1,), (0,)), ((), ())),
                               preferred_element_type=jnp.float32)
    cex = csum - cnt
    counts = csum[:, T - 1:T]
    padded = jnp.floor((counts + (TILE - 1)) * (1.0 / TILE)) * TILE
    e_i = jax.lax.broadcasted_iota(jnp.int32, (NE, NE), 0)
    e_j = jax.lax.broadcasted_iota(jnp.int32, (NE, NE), 1)
    lex = (e_j < e_i).astype(jnp.float32)
    offs = jax.lax.dot_general(lex, padded, (((1,), (0,)), ((), ())),
                               preferred_element_type=jnp.float32)
    pos_base = offs + cex
    p0 = jnp.sum(jnp.where(mask1, pos_base, 0.0), axis=0, keepdims=True)
    p1 = jnp.sum(jnp.where(mask2, pos_base, 0.0), axis=0, keepdims=True)
    p0_ref[...] = p0.astype(jnp.int32)
    p1_ref[...] = p1.astype(jnp.int32)

    jt = jax.lax.broadcasted_iota(jnp.int32, (NE, NT), 1).astype(jnp.float32)
    e_col = jax.lax.broadcasted_iota(jnp.int32, (NE, NT), 0)
    lo = offs * (1.0 / TILE)
    hi = (offs + padded) * (1.0 / TILE)
    sel = (jt >= lo) & (jt < hi)
    te = jnp.sum(jnp.where(sel, e_col, 0), axis=0, keepdims=True)
    jt_i = jax.lax.broadcasted_iota(jnp.int32, (1, NT), 1)
    te_ref[...] = jnp.where(jt_i >= NT_R, NE, te)


@jax.jit
def _stage_a(x2d, gate_weight):
    outs = pl.pallas_call(
        _router_body,
        grid=(1,),
        in_specs=[
            pl.BlockSpec((T, HID), lambda i: (0, 0)),
            pl.BlockSpec((NE, HID), lambda i: (0, 0)),
        ],
        out_specs=[
            pl.BlockSpec((T, 128), lambda i: (0, 0)),
            pl.BlockSpec((T, 128), lambda i: (0, 0)),
            pl.BlockSpec((1, T), lambda i: (0, 0)),
            pl.BlockSpec((1, T), lambda i: (0, 0)),
            pl.BlockSpec((1, NT), lambda i: (0, 0)),
        ],
        out_shape=[
            jax.ShapeDtypeStruct((T, 128), jnp.float32),
            jax.ShapeDtypeStruct((T, 128), jnp.float32),
            jax.ShapeDtypeStruct((1, T), jnp.int32),
            jax.ShapeDtypeStruct((1, T), jnp.int32),
            jax.ShapeDtypeStruct((1, NT), jnp.int32),
        ],
    )(x2d, gate_weight)
    w0, w1, p0, p1, te = outs
    return (w0, w1, p0.reshape(T), p1.reshape(T), te.reshape(NT))


def _scatter_body(x_hbm, p0_hbm, p1_hbm, w0_hbm, w1_hbm, xs_hbm, ws_hbm,
                  xrows, p0v, p1v, w0r, w1r, sem):
    wid = lax.axis_index("s") * 2 + lax.axis_index("c")
    base = wid * CHUNK_B
    pltpu.sync_copy(x_hbm.at[pl.ds(base, CHUNK_B)], xrows)
    pltpu.sync_copy(p0_hbm.at[pl.ds(base, CHUNK_B)], p0v)
    pltpu.sync_copy(p1_hbm.at[pl.ds(base, CHUNK_B)], p1v)
    pltpu.sync_copy(w0_hbm.at[pl.ds(base, CHUNK_B)], w0r)
    pltpu.sync_copy(w1_hbm.at[pl.ds(base, CHUNK_B)], w1r)
    a = pltpu.async_copy(xrows, xs_hbm.at[p0v], sem)
    b = pltpu.async_copy(xrows, xs_hbm.at[p1v], sem)
    c = pltpu.async_copy(w0r, ws_hbm.at[p0v], sem)
    d = pltpu.async_copy(w1r, ws_hbm.at[p1v], sem)
    a.wait()
    b.wait()
    c.wait()
    d.wait()


@jax.jit
def _stage_b(x2d, p0, p1, w0, w1):
    mesh = plsc.VectorSubcoreMesh(core_axis_name="c", subcore_axis_name="s")
    return pl.kernel(
        _scatter_body,
        mesh=mesh,
        out_type=[
            jax.ShapeDtypeStruct((S_PAD, HID), jnp.float32),
            jax.ShapeDtypeStruct((S_PAD, 128), jnp.float32),
        ],
        scratch_types=[
            pltpu.VMEM((CHUNK_B, HID), jnp.float32),
            pltpu.VMEM((CHUNK_B,), jnp.int32),
            pltpu.VMEM((CHUNK_B,), jnp.int32),
            pltpu.VMEM((CHUNK_B, 128), jnp.float32),
            pltpu.VMEM((CHUNK_B, 128), jnp.float32),
            pltpu.SemaphoreType.DMA,
        ],
    )(x2d, p0, p1, w0, w1)


def _gmm_body(te_ref, xs_ref, ws_ref, egw_ref, euw_ref, edw_ref, os_ref):
    j = pl.program_id(0)
    e = te_ref[j]
    xb = xs_ref[...]
    g = _dotT(xb, egw_ref[e])
    u = _dotT(xb, euw_ref[e])
    h = _silu(g) * u
    os_ref[...] = _dotT(h, edw_ref[e]) * ws_ref[:, 0:1]


@jax.jit
def _stage_c1(te, xs, ws, egw, euw, edw):
    grid_spec = pltpu.PrefetchScalarGridSpec(
        num_scalar_prefetch=1,
        grid=(NT_R,),
        in_specs=[
            pl.BlockSpec((TILE, HID), lambda j, s: (j, 0)),
            pl.BlockSpec((TILE, 128), lambda j, s: (j, 0)),
            pl.BlockSpec((NE, INT, HID), lambda j, s: (0, 0, 0)),
            pl.BlockSpec((NE, INT, HID), lambda j, s: (0, 0, 0)),
            pl.BlockSpec((NE, HID, INT), lambda j, s: (0, 0, 0)),
        ],
        out_specs=pl.BlockSpec((TILE, HID), lambda j, s: (j, 0)),
    )
    return pl.pallas_call(
        _gmm_body,
        grid_spec=grid_spec,
        out_shape=jax.ShapeDtypeStruct((S_PAD, HID), jnp.float32),
    )(te, xs, ws, egw, euw, edw)


STILE = 512


def _shared_body(x_ref, sgw_ref, suw_ref, sdw_ref, os_ref):
    xb = x_ref[...]
    g = _dotT(xb, sgw_ref[...])
    u = _dotT(xb, suw_ref[...])
    h = _silu(g) * u
    os_ref[...] = _dotT(h, sdw_ref[...])


@jax.jit
def _stage_c2(x2d, sgw, suw, sdw):
    return pl.pallas_call(
        _shared_body,
        grid=(T // STILE,),
        in_specs=[
            pl.BlockSpec((STILE, HID), lambda t: (t, 0)),
            pl.BlockSpec((INT, HID), lambda t: (0, 0)),
            pl.BlockSpec((INT, HID), lambda t: (0, 0)),
            pl.BlockSpec((HID, INT), lambda t: (0, 0)),
        ],
        out_specs=pl.BlockSpec((STILE, HID), lambda t: (t, 0)),
        out_shape=jax.ShapeDtypeStruct((T, HID), jnp.float32),
    )(x2d, sgw, suw, sdw)


def _combine_body(os_hbm, sh_hbm, p0_hbm, p1_hbm, out_hbm,
                  p0v, p1v, r0, r1, rs, sem):
    wid = lax.axis_index("s") * 2 + lax.axis_index("c")

    for c in range(CHUNK_B // CHUNK_D):
        base = wid * CHUNK_B + c * CHUNK_D
        pltpu.sync_copy(p0_hbm.at[pl.ds(base, CHUNK_D)], p0v)
        pltpu.sync_copy(p1_hbm.at[pl.ds(base, CHUNK_D)], p1v)
        a = pltpu.async_copy(os_hbm.at[p0v], r0, sem)
        b = pltpu.async_copy(os_hbm.at[p1v], r1, sem)
        pltpu.sync_copy(sh_hbm.at[pl.ds(base, CHUNK_D)], rs)
        a.wait()
        b.wait()

        def tok_body(i, carry):
            def col_body(k, carry2):
                for u in range(4):
                    sl = pl.ds((k * 4 + u) * 16, 16)
                    r0[i, sl] = r0[i, sl] + r1[i, sl] + rs[i, sl]
                return carry2

            return lax.fori_loop(0, HID // 64, col_body, carry)

        lax.fori_loop(0, CHUNK_D, tok_body, 0)
        pltpu.sync_copy(r0, out_hbm.at[pl.ds(base, CHUNK_D)])


@jax.jit
def _stage_d(osorted, shared_out, p0, p1):
    mesh = plsc.VectorSubcoreMesh(core_axis_name="c", subcore_axis_name="s")
    return pl.kernel(
        _combine_body,
        mesh=mesh,
        out_type=jax.ShapeDtypeStruct((T, HID), jnp.float32),
        scratch_types=[
            pltpu.VMEM((CHUNK_D,), jnp.int32),
            pltpu.VMEM((CHUNK_D,), jnp.int32),
            pltpu.VMEM((CHUNK_D, HID), jnp.float32),
            pltpu.VMEM((CHUNK_D, HID), jnp.float32),
            pltpu.VMEM((CHUNK_D, HID), jnp.float32),
            pltpu.SemaphoreType.DMA,
        ],
    )(osorted, shared_out, p0, p1)


def kernel(hidden_states, gate_weight, e_score_correction_bias,
           expert_gate_w, expert_up_w, expert_down_w,
           shared_gate_w, shared_up_w, shared_down_w):
    orig_shape = hidden_states.shape
    x2d = hidden_states.reshape(-1, orig_shape[-1])
    w0, w1, p0, p1, te = _stage_a(x2d, gate_weight)
    xs, ws = _stage_b(x2d, p0, p1, w0, w1)
    osorted = _stage_c1(te, xs, ws, expert_gate_w, expert_up_w,
                        expert_down_w)
    shared_out = _stage_c2(x2d, shared_gate_w, shared_up_w, shared_down_w)
    out = _stage_d(osorted, shared_out, p0, p1)
    return out.reshape(orig_shape)
